# baseline (speedup 1.0000x reference)
import jax
import jax.numpy as jnp
from jax import lax
from jax.experimental import pallas as pl
from jax.experimental.pallas import tpu as pltpu
from jax.experimental.pallas import tpu_sc as plsc

BATCH = 4096
SEQ = 20
DIM = 64
VOCAB = 100000
NUM_CORES = 2
NUM_SUBCORES = 16
NW = NUM_CORES * NUM_SUBCORES
B_PER_W = BATCH // NW
GB = 8
N_CHUNKS = B_PER_W // GB
IDS_PER_CHUNK = GB * SEQ
ROWS_PER_W = B_PER_W * SEQ
OUT_W = 2 * SEQ * DIM

TBLK = 1024
TGRID = (VOCAB + TBLK - 1) // TBLK
VPAD = TGRID * TBLK


def _pack_body(tt_ref, out_ref):
  ident = (lax.broadcasted_iota(jnp.int32, (128, 128), 0) ==
           lax.broadcasted_iota(jnp.int32, (128, 128), 1)
           ).astype(jnp.float32)
  out_ref[...] = lax.dot_general(
      tt_ref[...], ident, (((0,), (0,)), ((), ())),
      preferred_element_type=jnp.float32)


def _pack_table(table):
  return pl.pallas_call(
      _pack_body,
      grid=(TGRID,),
      in_specs=[pl.BlockSpec((128, TBLK), lambda i: (0, i))],
      out_specs=pl.BlockSpec((TBLK, 128), lambda i: (i, 0)),
      out_shape=jax.ShapeDtypeStruct((VPAD, 128), jnp.float32),
  )(table.T)


def _sc_body(ids0_hbm, ids1_hbm, len0_hbm, len1_hbm, t0_hbm, t1_hbm, out_hbm,
             sidx_v, idx0f, idx1f, lf0, lf1, len_v, g0_v, g1_v, c_v,
             gsem0, gsem1, wsem):
  cid = lax.axis_index("c")
  sid = lax.axis_index("s")
  wid = sid * NUM_CORES + cid
  b0w = wid * B_PER_W

  lrow = pl.multiple_of(wid - lax.rem(wid, 8), 8)
  lloc = lax.rem(wid, 8)
  pltpu.sync_copy(len0_hbm.at[pl.ds(lrow, 8)], len_v.at[pl.ds(0, 8)])
  pltpu.sync_copy(len1_hbm.at[pl.ds(lrow, 8)], len_v.at[pl.ds(8, 8)])
  for grp in range(8):
    lf0[pl.ds(grp * 16, 16)] = len_v[lloc, pl.ds(grp * 16, 16)]
    lf1[pl.ds(grp * 16, 16)] = len_v[lloc + 8, pl.ds(grp * 16, 16)]

  pad = lax.rem(wid, 2) * 4
  astart = pl.multiple_of(wid * 20 - pad, 8)
  pltpu.sync_copy(ids0_hbm.at[pl.ds(astart, 24)], sidx_v)
  for r in range(24):
    for cc in range(8):
      idx0f[pl.ds(r * 128 + cc * 16, 16)] = sidx_v[r, pl.ds(cc * 16, 16)]
  pltpu.sync_copy(ids1_hbm.at[pl.ds(astart, 24)], sidx_v)
  for r in range(24):
    for cc in range(8):
      idx1f[pl.ds(r * 128 + cc * 16, 16)] = sidx_v[r, pl.ds(cc * 16, 16)]
  loc0 = pad * 128

  zero = jnp.zeros((16,), jnp.float32)
  gbufs = (g0_v, g1_v)
  gsems = (gsem0, gsem1)

  def fire(c, p):
    g_v = gbufs[p]
    base = loc0 + c * IDS_PER_CHUNK
    for f, idxf, t_hbm in ((0, idx0f, t0_hbm), (1, idx1f, t1_hbm)):
      for off, n in ((0, 128), (128, 32)):
        pltpu.async_copy(t_hbm.at[idxf.at[pl.ds(base + off, n)]],
                         g_v.at[pl.ds(f * IDS_PER_CHUNK + off, n)], gsems[p])

  def drain_g(p):
    pltpu.make_async_copy(t0_hbm.at[pl.ds(0, 2 * IDS_PER_CHUNK)], gbufs[p],
                          gsems[p]).wait()

  def drain_w():
    pltpu.make_async_copy(out_hbm.at[pl.ds(0, GB)], c_v, wsem).wait()

  def chunk_work(c2, e):
    c = 2 * c2 + e
    g_v = gbufs[e]
    drain_g(e)
    lv0 = lf0[pl.ds(c2 * 16, 16)]
    lv1 = lf1[pl.ds(c2 * 16, 16)]
    for f, lv in ((0, lv0), (1, lv1)):
      for bi in range(GB):
        base_row = f * IDS_PER_CHUNK + bi * SEQ

        def zrow(s, _):
          r = base_row + s
          g_v[r, pl.ds(0, 16)] = zero
          g_v[r, pl.ds(16, 16)] = zero
          g_v[r, pl.ds(32, 16)] = zero
          g_v[r, pl.ds(48, 16)] = zero
          return 0

        lax.fori_loop(lv[e * 8 + bi], SEQ, zrow, 0)
    @pl.when(c > 0)
    def _():
      drain_w()

    for bi in range(GB):
      for j in range(2 * SEQ):
        src_row = (bi * SEQ + j) if j < SEQ else (IDS_PER_CHUNK + bi * SEQ
                                                  + (j - SEQ))
        for k in range(DIM // 16):
          c_v[bi, pl.ds(j * DIM + k * 16, 16)] = g_v[src_row,
                                                     pl.ds(k * 16, 16)]
    row0 = pl.multiple_of(b0w + c * GB, 8)
    pltpu.async_copy(c_v, out_hbm.at[pl.ds(row0, GB)], wsem)

    @pl.when(c < N_CHUNKS - 2)
    def _():
      fire(c + 2, e)

  fire(0, 0)
  fire(1, 1)

  def pair_main(c2, carry):
    chunk_work(c2, 0)
    chunk_work(c2, 1)
    return carry

  lax.fori_loop(0, N_CHUNKS // 2, pair_main, 0)
  drain_w()


@jax.jit
def _run(ids_f0, ids_f1, lengths_f0, lengths_f1, table_f0, table_f1):
  t0p = _pack_table(table_f0)
  t1p = _pack_table(table_f1)
  mesh = plsc.VectorSubcoreMesh(core_axis_name="c", subcore_axis_name="s")
  ids0 = ids_f0.reshape(BATCH * SEQ // 128, 128)
  ids1 = ids_f1.reshape(BATCH * SEQ // 128, 128)
  l0 = lengths_f0.reshape(32, 128)
  l1 = lengths_f1.reshape(32, 128)
  out = pl.kernel(
      _sc_body,
      out_type=jax.ShapeDtypeStruct((BATCH, OUT_W), jnp.float32),
      mesh=mesh,
      compiler_params=pltpu.CompilerParams(use_tc_tiling_on_sc=True),
      scratch_types=[
          pltpu.VMEM((24, 128), jnp.int32),
          pltpu.VMEM((3072,), jnp.int32),
          pltpu.VMEM((3072,), jnp.int32),
          pltpu.VMEM((B_PER_W,), jnp.int32),
          pltpu.VMEM((B_PER_W,), jnp.int32),
          pltpu.VMEM((16, 128), jnp.int32),
          pltpu.VMEM((2 * IDS_PER_CHUNK, 128), jnp.float32),
          pltpu.VMEM((2 * IDS_PER_CHUNK, 128), jnp.float32),
          pltpu.VMEM((GB, OUT_W), jnp.float32),
          pltpu.SemaphoreType.DMA,
          pltpu.SemaphoreType.DMA,
          pltpu.SemaphoreType.DMA,
      ],
  )(ids0, ids1, l0, l1, t0p, t1p)
  return out


def kernel(ids_f0, ids_f1, lengths_f0, lengths_f1, table_f0, table_f1):
  return _run(ids_f0, ids_f1, lengths_f0, lengths_f1, table_f0, table_f1)


# speedup vs baseline: 1.2810x; 1.2810x over previous
import jax
import jax.numpy as jnp
from jax import lax
from jax.experimental import pallas as pl
from jax.experimental.pallas import tpu as pltpu
from jax.experimental.pallas import tpu_sc as plsc

BATCH = 4096
SEQ = 20
DIM = 64
VOCAB = 100000
NUM_CORES = 2
NUM_SUBCORES = 16
NW = NUM_CORES * NUM_SUBCORES
B_PER_W = BATCH // NW
GB = 8
N_CHUNKS = B_PER_W // GB
IDS_PER_CHUNK = GB * SEQ
ROWS_PER_W = B_PER_W * SEQ
OUT_W = 2 * SEQ * DIM

def _pack_table(table):
  return jnp.pad(table, ((0, 0), (0, 128 - DIM)))


def _sc_body(ids0_hbm, ids1_hbm, len0_hbm, len1_hbm, t0_hbm, t1_hbm, out_hbm,
             sidx_v, idx0f, idx1f, lf0, lf1, len_v, g0_v, g1_v, c_v,
             gsem0, gsem1, wsem):
  cid = lax.axis_index("c")
  sid = lax.axis_index("s")
  wid = sid * NUM_CORES + cid
  b0w = wid * B_PER_W

  lrow = pl.multiple_of(wid - lax.rem(wid, 8), 8)
  lloc = lax.rem(wid, 8)
  pltpu.sync_copy(len0_hbm.at[pl.ds(lrow, 8)], len_v.at[pl.ds(0, 8)])
  pltpu.sync_copy(len1_hbm.at[pl.ds(lrow, 8)], len_v.at[pl.ds(8, 8)])
  for grp in range(8):
    lf0[pl.ds(grp * 16, 16)] = len_v[lloc, pl.ds(grp * 16, 16)]
    lf1[pl.ds(grp * 16, 16)] = len_v[lloc + 8, pl.ds(grp * 16, 16)]

  pad = lax.rem(wid, 2) * 4
  astart = pl.multiple_of(wid * 20 - pad, 8)
  pltpu.sync_copy(ids0_hbm.at[pl.ds(astart, 24)], sidx_v)
  for r in range(24):
    for cc in range(8):
      idx0f[pl.ds(r * 128 + cc * 16, 16)] = sidx_v[r, pl.ds(cc * 16, 16)]
  pltpu.sync_copy(ids1_hbm.at[pl.ds(astart, 24)], sidx_v)
  for r in range(24):
    for cc in range(8):
      idx1f[pl.ds(r * 128 + cc * 16, 16)] = sidx_v[r, pl.ds(cc * 16, 16)]
  loc0 = pad * 128

  zero = jnp.zeros((16,), jnp.float32)
  gbufs = (g0_v, g1_v)
  gsems = (gsem0, gsem1)

  def fire(c, p):
    g_v = gbufs[p]
    base = loc0 + c * IDS_PER_CHUNK
    for f, idxf, t_hbm in ((0, idx0f, t0_hbm), (1, idx1f, t1_hbm)):
      for off, n in ((0, 128), (128, 32)):
        pltpu.async_copy(t_hbm.at[idxf.at[pl.ds(base + off, n)]],
                         g_v.at[pl.ds(f * IDS_PER_CHUNK + off, n)], gsems[p])

  def drain_g(p):
    pltpu.make_async_copy(t0_hbm.at[pl.ds(0, 2 * IDS_PER_CHUNK)], gbufs[p],
                          gsems[p]).wait()

  def drain_w():
    pltpu.make_async_copy(out_hbm.at[pl.ds(0, GB)], c_v, wsem).wait()

  def chunk_work(c2, e):
    c = 2 * c2 + e
    g_v = gbufs[e]
    drain_g(e)
    lv0 = lf0[pl.ds(c2 * 16, 16)]
    lv1 = lf1[pl.ds(c2 * 16, 16)]
    for f, lv in ((0, lv0), (1, lv1)):
      for bi in range(GB):
        base_row = f * IDS_PER_CHUNK + bi * SEQ

        def zrow(s, _):
          r = base_row + s
          g_v[r, pl.ds(0, 16)] = zero
          g_v[r, pl.ds(16, 16)] = zero
          g_v[r, pl.ds(32, 16)] = zero
          g_v[r, pl.ds(48, 16)] = zero
          return 0

        lax.fori_loop(lv[e * 8 + bi], SEQ, zrow, 0)
    @pl.when(c > 0)
    def _():
      drain_w()

    for bi in range(GB):
      for j in range(2 * SEQ):
        src_row = (bi * SEQ + j) if j < SEQ else (IDS_PER_CHUNK + bi * SEQ
                                                  + (j - SEQ))
        for k in range(DIM // 16):
          c_v[bi, pl.ds(j * DIM + k * 16, 16)] = g_v[src_row,
                                                     pl.ds(k * 16, 16)]
    row0 = pl.multiple_of(b0w + c * GB, 8)
    pltpu.async_copy(c_v, out_hbm.at[pl.ds(row0, GB)], wsem)

    @pl.when(c < N_CHUNKS - 2)
    def _():
      fire(c + 2, e)

  fire(0, 0)
  fire(1, 1)

  def pair_main(c2, carry):
    chunk_work(c2, 0)
    chunk_work(c2, 1)
    return carry

  lax.fori_loop(0, N_CHUNKS // 2, pair_main, 0)
  drain_w()


@jax.jit
def _run(ids_f0, ids_f1, lengths_f0, lengths_f1, table_f0, table_f1):
  t0p = _pack_table(table_f0)
  t1p = _pack_table(table_f1)
  mesh = plsc.VectorSubcoreMesh(core_axis_name="c", subcore_axis_name="s")
  ids0 = ids_f0.reshape(BATCH * SEQ // 128, 128)
  ids1 = ids_f1.reshape(BATCH * SEQ // 128, 128)
  l0 = lengths_f0.reshape(32, 128)
  l1 = lengths_f1.reshape(32, 128)
  out = pl.kernel(
      _sc_body,
      out_type=jax.ShapeDtypeStruct((BATCH, OUT_W), jnp.float32),
      mesh=mesh,
      compiler_params=pltpu.CompilerParams(use_tc_tiling_on_sc=True),
      scratch_types=[
          pltpu.VMEM((24, 128), jnp.int32),
          pltpu.VMEM((3072,), jnp.int32),
          pltpu.VMEM((3072,), jnp.int32),
          pltpu.VMEM((B_PER_W,), jnp.int32),
          pltpu.VMEM((B_PER_W,), jnp.int32),
          pltpu.VMEM((16, 128), jnp.int32),
          pltpu.VMEM((2 * IDS_PER_CHUNK, 128), jnp.float32),
          pltpu.VMEM((2 * IDS_PER_CHUNK, 128), jnp.float32),
          pltpu.VMEM((GB, OUT_W), jnp.float32),
          pltpu.SemaphoreType.DMA,
          pltpu.SemaphoreType.DMA,
          pltpu.SemaphoreType.DMA,
      ],
  )(ids0, ids1, l0, l1, t0p, t1p)
  return out


def kernel(ids_f0, ids_f1, lengths_f0, lengths_f1, table_f0, table_f1):
  return _run(ids_f0, ids_f1, lengths_f0, lengths_f1, table_f0, table_f1)
